# baseline (speedup 1.0000x reference)
import functools

import jax
import jax.numpy as jnp
from jax import lax
from jax.experimental import pallas as pl
from jax.experimental.pallas import tpu as pltpu
from jax.experimental.pallas import tpu_sc as plsc

N = 10000
E = 160000
TE = 2 * E
T = 2
C = 2
D = 128
NUM_LAYERS = 2
BETA = 0.5
TP = 0.1

NC = 2
NS = 16
K = 64
NSUB = 32
SUP = NSUB * K
EP = 163840
TEP = 2 * EP
EPT = TEP // NS
NSUP = EPT // SUP
NP = 10240
RPT = NP // NS
ZR = 32
LANES = 16


def _mm_body(x_ref, w_ref, o_ref):
    o_ref[0] = jnp.dot(x_ref[...], w_ref[0], preferred_element_type=jnp.float32)


def _channel_matmul(X, Ws):
    BN = 400
    return pl.pallas_call(
        _mm_body,
        grid=(C, N // BN),
        in_specs=[
            pl.BlockSpec((BN, D), lambda c, i: (i, 0)),
            pl.BlockSpec((1, D, D), lambda c, i: (c, 0, 0)),
        ],
        out_specs=pl.BlockSpec((1, BN, D), lambda c, i: (c, i, 0)),
        out_shape=jax.ShapeDtypeStruct((C, NP, D), jnp.float32),
    )(X, Ws)


def _sc_body(h0_hbm, rows_hbm, cols_hbm, vals_hbm, lw_hbm,
             h2_hbm, h1_hbm,
             ridx2, cidx, vbuf, rb0, rb1, rb2, rb3, zbuf, lwbuf, acc,
             gsem0, gsem1, gsem2, gsem3, ssem0, ssem1, ssem2, ssem3):
    c = lax.axis_index("c")
    s = lax.axis_index("s")
    ttype = s // (NS // T)
    rbase = s * RPT
    ebase = s * EPT

    rbufs = (rb0, rb1, rb2, rb3)
    gsems = (gsem0, gsem1, gsem2, gsem3)
    ssems = (ssem0, ssem1, ssem2, ssem3)

    def _zrow(r, _):
        for j in range(D // LANES):
            zbuf[r, pl.ds(j * LANES, LANES)] = jnp.zeros((LANES,), jnp.float32)
        return 0
    lax.fori_loop(0, ZR, _zrow, 0)
    for z in range(RPT // ZR):
        pltpu.sync_copy(zbuf, acc.at[pl.ds(rbase + z * ZR, ZR)])
    pltpu.sync_copy(lw_hbm, lwbuf.at[pl.ds(0, LANES)])
    plsc.subcore_barrier()

    lwbuf[pl.ds(LANES, LANES)] = jnp.exp(lwbuf[pl.ds(0, LANES)])

    def _expw(i):
        return lwbuf[pl.ds(LANES + i, LANES)][0]

    for l in range(NUM_LAYERS):
        src = h0_hbm if l == 0 else h1_hbm
        dst = h1_hbm if l == 0 else h2_hbm
        base = l * (C * T) + c * T
        e0 = jnp.full((LANES,), _expw(base))
        e1 = jnp.full((LANES,), _expw(base + 1))
        scale = jnp.where(ttype == 0, e0, e1) / (e0 + e1)

        def _super(sp, _):
            off = ebase + sp * SUP
            pltpu.sync_copy(
                rows_hbm.at[pl.ds(pl.multiple_of(off // K, 32), NSUB)], ridx2)
            pltpu.sync_copy(cols_hbm.at[pl.ds(c * TEP + off, SUP)], cidx)
            pltpu.sync_copy(vals_hbm.at[pl.ds(off, SUP)], vbuf.at[pl.ds(0, SUP)])

            NB = 4

            def _gather(j):
                b = j % NB
                return pltpu.async_copy(
                    src.at[cidx.at[pl.ds(j * K, K)]], rbufs[b], gsems[b])

            gd = [None] * NB
            sd = [None] * NB
            for j0 in range(NB - 1):
                gd[j0] = _gather(j0)
            for j in range(NSUB):
                b = j % NB
                fb = (j + NB - 1) % NB
                if j + NB - 1 < NSUB:
                    if sd[fb] is not None:
                        sd[fb].wait()
                        sd[fb] = None
                    gd[fb] = _gather(j + NB - 1)
                gd[b].wait()

                rb = rbufs[b]
                joff = j * K

                @plsc.parallel_loop(0, K, 1, unroll=4)
                def _row(r):
                    v = vbuf[pl.ds(joff + r, LANES)][0] * scale
                    for q in range(D // LANES):
                        sl = rb[r, pl.ds(q * LANES, LANES)]
                        rb[r, pl.ds(q * LANES, LANES)] = sl * v
                sd[b] = pltpu.async_copy(rb, acc.at[ridx2.at[j]], ssems[b],
                                         add=True)
            for b in range(NB):
                if sd[b] is not None:
                    sd[b].wait()
            return 0
        lax.fori_loop(0, NSUP, _super, 0)

        plsc.subcore_barrier()
        pltpu.sync_copy(acc.at[pl.ds(rbase, RPT)],
                        dst.at[pl.ds(c * NP + rbase, RPT)])
        if l < NUM_LAYERS - 1:
            for z in range(RPT // ZR):
                pltpu.sync_copy(zbuf, acc.at[pl.ds(rbase + z * ZR, ZR)])
        plsc.subcore_barrier()


def _sc_spmm(h0f, rows, cols2, vals, lw16):
    mesh = plsc.VectorSubcoreMesh(core_axis_name="c", subcore_axis_name="s",
                                  num_cores=NC, num_subcores=NS)
    fn = pl.kernel(
        _sc_body,
        out_type=(
            jax.ShapeDtypeStruct((C * NP, D), jnp.float32),
            jax.ShapeDtypeStruct((C * NP, D), jnp.float32),
        ),
        mesh=mesh,
        scratch_types=[
            pltpu.VMEM((NSUB, K), jnp.int32),
            pltpu.VMEM((SUP,), jnp.int32),
            pltpu.VMEM((SUP + LANES,), jnp.float32),
            pltpu.VMEM((K, D), jnp.float32),
            pltpu.VMEM((K, D), jnp.float32),
            pltpu.VMEM((K, D), jnp.float32),
            pltpu.VMEM((K, D), jnp.float32),
            pltpu.VMEM((ZR, D), jnp.float32),
            pltpu.VMEM((3 * LANES,), jnp.float32),
            pltpu.VMEM_SHARED((NP, D), jnp.float32),
            pltpu.SemaphoreType.DMA,
            pltpu.SemaphoreType.DMA,
            pltpu.SemaphoreType.DMA,
            pltpu.SemaphoreType.DMA,
            pltpu.SemaphoreType.DMA,
            pltpu.SemaphoreType.DMA,
            pltpu.SemaphoreType.DMA,
            pltpu.SemaphoreType.DMA,
        ],
    )
    h2f, _ = fn(h0f, rows, cols2, vals, lw16)
    return h2f


def _ep_body(x_ref, h_ref, w_ref, b_ref, o_ref):
    acc = jnp.broadcast_to(b_ref[0], o_ref.shape).astype(jnp.float32)
    for c in range(C):
        xc = x_ref[c]
        hc = h_ref[c]
        g = TP * jnp.maximum(BETA * xc + (1.0 - BETA) * hc, 0.0) + (1.0 - TP) * xc
        acc = acc + jnp.dot(g, w_ref[c], preferred_element_type=jnp.float32)
    o_ref[...] = jnp.maximum(acc, 0.0)


def _epilogue(H0, H2, lin_W, lin_b):
    BN = 400
    return pl.pallas_call(
        _ep_body,
        grid=(N // BN,),
        in_specs=[
            pl.BlockSpec((C, BN, D), lambda i: (0, i, 0)),
            pl.BlockSpec((C, BN, D), lambda i: (0, i, 0)),
            pl.BlockSpec((C, D, D), lambda i: (0, 0, 0)),
            pl.BlockSpec((1, D), lambda i: (0, 0)),
        ],
        out_specs=pl.BlockSpec((BN, D), lambda i: (i, 0)),
        out_shape=jax.ShapeDtypeStruct((N, D), jnp.float32),
    )(H0, H2, lin_W, lin_b)


def kernel(A0_index, A0_value, A1_index, A1_value, X, Ws, layer_weights, lin_W, lin_b):
    padi = jnp.full((EP - E,), NP - 1, jnp.int32)
    padc = jnp.zeros((EP - E,), jnp.int32)
    padv = jnp.zeros((EP - E,), jnp.float32)
    rows = jnp.concatenate([A0_index[0].astype(jnp.int32), padi,
                            A1_index[0].astype(jnp.int32), padi])
    cols = jnp.concatenate([A0_index[1].astype(jnp.int32), padc,
                            A1_index[1].astype(jnp.int32), padc])
    cols2 = jnp.concatenate([cols, cols + NP])
    rows2 = rows.reshape(TEP // K, K)
    vals = jnp.concatenate([A0_value, padv, A1_value, padv])
    lw16 = jnp.pad(layer_weights.reshape(-1), (0, LANES - NUM_LAYERS * C * T))

    H0 = _channel_matmul(X, Ws)
    h2f = _sc_spmm(H0.reshape(C * NP, D), rows2, cols2, vals, lw16)
    Wr = lin_W.reshape(C, D, D)
    return _epilogue(H0, h2f.reshape(C, NP, D), Wr, lin_b.reshape(1, D))
